# Initial kernel scaffold; baseline (speedup 1.0000x reference)
import jax
import jax.numpy as jnp
from jax.experimental import pallas as pl


def kernel(x, table):
    raise NotImplementedError("write your pallas kernel here")


# speedup vs baseline: 4.8038x; 4.8038x over previous
import functools

import jax
import jax.numpy as jnp
from jax import lax
from jax.experimental import pallas as pl
from jax.experimental.pallas import tpu as pltpu
from jax.experimental.pallas import tpu_sc as plsc

DIM = 32
NUM_CORES = 2
NUM_SUBCORES = 16
NUM_WORKERS = NUM_CORES * NUM_SUBCORES
CHUNK = 1024
GATHER_SPLIT = 128


@functools.partial(jax.jit, static_argnames=())
def _sc_embedding_gather(x_flat, table):
    total = x_flat.shape[0]
    per_worker = total // NUM_WORKERS
    n_chunks = per_worker // CHUNK
    mesh = plsc.VectorSubcoreMesh(core_axis_name="c", subcore_axis_name="s")

    @functools.partial(
        pl.kernel,
        mesh=mesh,
        out_type=jax.ShapeDtypeStruct((total, DIM), jnp.float32),
        scratch_types=[
            pltpu.VMEM((CHUNK,), jnp.int32),
            pltpu.VMEM((CHUNK, DIM), jnp.float32),
            pltpu.SemaphoreType.DMA,
        ],
        compiler_params=pltpu.CompilerParams(use_tc_tiling_on_sc=False),
    )
    def k(idx_hbm, table_hbm, out_hbm, idx_v, rows_v, sem):
        wid = lax.axis_index("s") * NUM_CORES + lax.axis_index("c")
        base = wid * per_worker

        def body(i, carry):
            off = base + i * CHUNK
            pltpu.sync_copy(idx_hbm.at[pl.ds(off, CHUNK)], idx_v)
            copies = []
            for j in range(CHUNK // GATHER_SPLIT):
                copies.append(
                    pltpu.async_copy(
                        table_hbm.at[idx_v.at[pl.ds(j * GATHER_SPLIT, GATHER_SPLIT)]],
                        rows_v.at[pl.ds(j * GATHER_SPLIT, GATHER_SPLIT)],
                        sem,
                    )
                )
            for c in copies:
                c.wait()
            pltpu.sync_copy(rows_v, out_hbm.at[pl.ds(off, CHUNK)])
            return carry

        lax.fori_loop(0, n_chunks, body, 0)

    return k(x_flat, table)


def kernel(x, table):
    batch, hist = x.shape
    x_flat = x.reshape(-1).astype(jnp.int32)
    out = _sc_embedding_gather(x_flat, table)
    return out.reshape(batch, hist, DIM)


# speedup vs baseline: 5.0252x; 1.0461x over previous
import functools

import jax
import jax.numpy as jnp
from jax import lax
from jax.experimental import pallas as pl
from jax.experimental.pallas import tpu as pltpu
from jax.experimental.pallas import tpu_sc as plsc

DIM = 32
NUM_CORES = 2
NUM_SUBCORES = 16
NUM_WORKERS = NUM_CORES * NUM_SUBCORES
CHUNK = 1024
GATHER_SPLIT = 128
NBUF = 2


@jax.jit
def _sc_embedding_gather(x_flat, table):
    total = x_flat.shape[0]
    per_worker = total // NUM_WORKERS
    n_chunks = per_worker // CHUNK
    assert n_chunks % NBUF == 0 and n_chunks // NBUF >= 2
    mesh = plsc.VectorSubcoreMesh(core_axis_name="c", subcore_axis_name="s")

    @functools.partial(
        pl.kernel,
        mesh=mesh,
        out_type=jax.ShapeDtypeStruct((total, DIM), jnp.float32),
        scratch_types=[
            pltpu.VMEM((NBUF, CHUNK), jnp.int32),
            pltpu.VMEM((NBUF, CHUNK, DIM), jnp.float32),
            pltpu.SemaphoreType.DMA,
            pltpu.SemaphoreType.DMA,
            pltpu.SemaphoreType.DMA,
            pltpu.SemaphoreType.DMA,
            pltpu.SemaphoreType.DMA,
            pltpu.SemaphoreType.DMA,
        ],
        compiler_params=pltpu.CompilerParams(use_tc_tiling_on_sc=False),
    )
    def k(idx_hbm, table_hbm, out_hbm, idx_v, rows_v,
          gat0, gat1, out0, out1, lidx0, lidx1):
        gat = [gat0, gat1]
        osem = [out0, out1]
        isem = [lidx0, lidx1]
        wid = lax.axis_index("s") * NUM_CORES + lax.axis_index("c")
        base = wid * per_worker

        def load_idx(i, b):
            pltpu.async_copy(
                idx_hbm.at[pl.ds(base + i * CHUNK, CHUNK)], idx_v.at[b], isem[b])

        def wait_idx(b):
            pltpu.make_async_copy(
                idx_hbm.at[pl.ds(base, CHUNK)], idx_v.at[b], isem[b]).wait()

        def fire_gathers(i, b):
            for j in range(CHUNK // GATHER_SPLIT):
                sl = pl.ds(j * GATHER_SPLIT, GATHER_SPLIT)
                pltpu.async_copy(
                    table_hbm.at[idx_v.at[b, sl]], rows_v.at[b, sl], gat[b])

        def wait_gathers(b):
            for j in range(CHUNK // GATHER_SPLIT):
                sl = pl.ds(j * GATHER_SPLIT, GATHER_SPLIT)
                pltpu.make_async_copy(
                    table_hbm.at[idx_v.at[b, sl]], rows_v.at[b, sl], gat[b]).wait()

        def store_out(i, b):
            pltpu.async_copy(
                rows_v.at[b], out_hbm.at[pl.ds(base + i * CHUNK, CHUNK)], osem[b])

        def wait_out(b):
            pltpu.make_async_copy(
                rows_v.at[b], out_hbm.at[pl.ds(base, CHUNK)], osem[b]).wait()

        for b in range(NBUF):
            load_idx(b, b)
        for b in range(NBUF):
            wait_idx(b)
            fire_gathers(b, b)
        for b in range(NBUF):
            wait_gathers(b)
            store_out(b, b)
            load_idx(b + NBUF, b)

        def body(g, carry):
            i0 = g * NBUF
            for b in range(NBUF):
                wait_idx(b)
                wait_out(b)
                fire_gathers(i0 + b, b)
            for b in range(NBUF):
                wait_gathers(b)
                store_out(i0 + b, b)
                load_idx(i0 + b + NBUF, b)
            return carry

        lax.fori_loop(1, n_chunks // NBUF - 1, body, 0)

        i0 = n_chunks - NBUF
        for b in range(NBUF):
            wait_idx(b)
            wait_out(b)
            fire_gathers(i0 + b, b)
        for b in range(NBUF):
            wait_gathers(b)
            store_out(i0 + b, b)
        for b in range(NBUF):
            wait_out(b)

    return k(x_flat, table)


def kernel(x, table):
    batch, hist = x.shape
    x_flat = x.reshape(-1).astype(jnp.int32)
    out = _sc_embedding_gather(x_flat, table)
    return out.reshape(batch, hist, DIM)
